# baseline (speedup 1.0000x reference)
import functools

import jax
import jax.numpy as jnp
from jax import lax
from jax.experimental import pallas as pl
from jax.experimental.pallas import tpu as pltpu
from jax.experimental.pallas import tpu_sc as plsc

_N_NODES = 1000000
_N_UPD = 4194304
_ACC = 1 << 20
_N_TILES = 16
_W = 8192
_PER_TILE = _N_UPD // _N_TILES
_N_WIN = _PER_TILE // _W
_ACC_SLICE = _ACC // _N_TILES


def _sc_scatter(node_strengths, node_indices):
    mesh = plsc.VectorSubcoreMesh(core_axis_name="c", subcore_axis_name="s")

    @functools.partial(
        pl.kernel,
        out_type=(
            jax.ShapeDtypeStruct((_ACC,), jnp.float32),
            jax.ShapeDtypeStruct((_ACC,), jnp.float32),
        ),
        mesh=mesh,
        scratch_types=[
            pltpu.VMEM_SHARED((_ACC,), jnp.float32),
            pltpu.VMEM((_W,), jnp.int32),
            pltpu.VMEM((_W,), jnp.int32),
            pltpu.VMEM((_W,), jnp.float32),
            pltpu.VMEM((_W,), jnp.float32),
            pltpu.SemaphoreType.DMA,
            pltpu.SemaphoreType.DMA,
            pltpu.SemaphoreType.DMA,
            pltpu.SemaphoreType.DMA,
            pltpu.SemaphoreType.DMA,
            pltpu.SemaphoreType.DMA,
        ],
    )
    def scatter_kernel(vals_hbm, idx_hbm, sums_hbm, counts_hbm,
                       acc_sh, idx0, idx1, val0, val1,
                       sem_i0, sem_i1, sem_v0, sem_v1, sem_s0, sem_s1):
        cid = lax.axis_index("c")
        sid = lax.axis_index("s")
        upd_base = sid * _PER_TILE
        acc_base = sid * _ACC_SLICE

        def idx_copy(w, buf, sem):
            return pltpu.make_async_copy(
                idx_hbm.at[pl.ds(upd_base + w * _W, _W)], buf, sem)

        def val_copy(w, buf, sem):
            return pltpu.make_async_copy(
                vals_hbm.at[pl.ds(upd_base + w * _W, _W)], buf, sem)

        idx_copy(0, idx0, sem_i0).start()
        idx_copy(1, idx1, sem_i1).start()

        @pl.loop(0, _W, step=16)
        def _(i):
            val0[pl.ds(i, 16)] = jnp.zeros((16,), jnp.float32)

        @pl.loop(0, _ACC_SLICE, step=_W)
        def _(j):
            pltpu.async_copy(val0, acc_sh.at[pl.ds(acc_base + j, _W)],
                             sem_s0)

        @pl.loop(0, _ACC_SLICE, step=_W)
        def _(j):
            pltpu.make_async_copy(
                val0, acc_sh.at[pl.ds(acc_base + j, _W)], sem_s0).wait()

        @pl.when(cid == 0)
        def _():
            val_copy(0, val0, sem_v0).start()
            val_copy(1, val1, sem_v1).start()

        @pl.when(cid == 1)
        def _():
            @pl.loop(0, _W, step=16)
            def _(i):
                val0[pl.ds(i, 16)] = jnp.full((16,), 1.0, jnp.float32)

        plsc.subcore_barrier()

        @pl.loop(0, _N_WIN, step=2)
        def _(w):
            idx_copy(w, idx0, sem_i0).wait()

            @pl.when(cid == 0)
            def _():
                val_copy(w, val0, sem_v0).wait()
                pltpu.async_copy(val0, acc_sh.at[idx0], sem_s0, add=True)

            @pl.when(cid == 1)
            def _():
                pltpu.async_copy(val0, acc_sh.at[idx0], sem_s0, add=True)

            idx_copy(w + 1, idx1, sem_i1).wait()

            @pl.when(cid == 0)
            def _():
                val_copy(w + 1, val1, sem_v1).wait()
                pltpu.async_copy(val1, acc_sh.at[idx1], sem_s1, add=True)
                pltpu.make_async_copy(
                    val0, acc_sh.at[idx0], sem_s0).wait()

            @pl.when(cid == 1)
            def _():
                pltpu.async_copy(val0, acc_sh.at[idx1], sem_s1, add=True)
                pltpu.make_async_copy(
                    val0, acc_sh.at[idx0], sem_s0).wait()

            @pl.when(w + 2 < _N_WIN)
            def _():
                idx_copy(w + 2, idx0, sem_i0).start()

                @pl.when(cid == 0)
                def _():
                    val_copy(w + 2, val0, sem_v0).start()

            @pl.when(cid == 0)
            def _():
                pltpu.make_async_copy(
                    val1, acc_sh.at[idx1], sem_s1).wait()

            @pl.when(cid == 1)
            def _():
                pltpu.make_async_copy(
                    val0, acc_sh.at[idx1], sem_s1).wait()

            @pl.when(w + 3 < _N_WIN)
            def _():
                idx_copy(w + 3, idx1, sem_i1).start()

                @pl.when(cid == 0)
                def _():
                    val_copy(w + 3, val1, sem_v1).start()

        plsc.subcore_barrier()

        acc_slc = pl.ds(acc_base, _ACC_SLICE)

        @pl.when(cid == 0)
        def _():
            pltpu.sync_copy(acc_sh.at[acc_slc], sums_hbm.at[acc_slc])

        @pl.when(cid == 1)
        def _():
            pltpu.sync_copy(acc_sh.at[acc_slc], counts_hbm.at[acc_slc])

    return scatter_kernel(node_strengths, node_indices)


def _combine_body(s_ref, sum_ref, cnt_ref, o_ref):
    o_ref[...] = s_ref[...] + sum_ref[...] / jnp.maximum(cnt_ref[...], 1.0)


def _combine(strengths, sums, counts):
    blk = 131072
    return pl.pallas_call(
        _combine_body,
        out_shape=jax.ShapeDtypeStruct((_N_NODES,), jnp.float32),
        grid=(8,),
        in_specs=[pl.BlockSpec((blk,), lambda i: (i,))] * 3,
        out_specs=pl.BlockSpec((blk,), lambda i: (i,)),
    )(strengths, sums, counts)


def kernel(node_strengths, node_indices, strengths):
    sums, counts = _sc_scatter(node_strengths, node_indices)
    return _combine(strengths, sums, counts)


# speedup vs baseline: 1.2011x; 1.2011x over previous
import functools

import jax
import jax.numpy as jnp
from jax import lax
from jax.experimental import pallas as pl
from jax.experimental.pallas import tpu as pltpu
from jax.experimental.pallas import tpu_sc as plsc

_N_NODES = 1000000
_N_UPD = 4194304
_ACC = 1 << 20
_N_TILES = 16
_W = 8192
_PER_TILE = _N_UPD // _N_TILES
_N_WIN = _PER_TILE // _W
_ACC_SLICE = _ACC // _N_TILES


def _sc_scatter(node_strengths, node_indices):
    mesh = plsc.VectorSubcoreMesh(core_axis_name="c", subcore_axis_name="s")

    @functools.partial(
        pl.kernel,
        out_type=(
            jax.ShapeDtypeStruct((_ACC,), jnp.float32),
            jax.ShapeDtypeStruct((_ACC,), jnp.float32),
        ),
        mesh=mesh,
        scratch_types=[
            pltpu.VMEM_SHARED((_ACC,), jnp.float32),
            pltpu.VMEM((_W,), jnp.int32),
            pltpu.VMEM((_W,), jnp.int32),
            pltpu.VMEM((_W,), jnp.float32),
            pltpu.VMEM((_W,), jnp.float32),
            pltpu.SemaphoreType.DMA,
            pltpu.SemaphoreType.DMA,
            pltpu.SemaphoreType.DMA,
            pltpu.SemaphoreType.DMA,
            pltpu.SemaphoreType.DMA,
            pltpu.SemaphoreType.DMA,
        ],
    )
    def scatter_kernel(vals_hbm, idx_hbm, sums_hbm, counts_hbm,
                       acc_sh, idx0, idx1, val0, val1,
                       sem_i0, sem_i1, sem_v0, sem_v1, sem_s0, sem_s1):
        cid = lax.axis_index("c")
        sid = lax.axis_index("s")
        upd_base = sid * _PER_TILE
        acc_base = sid * _ACC_SLICE

        def idx_copy(w, buf, sem):
            return pltpu.make_async_copy(
                idx_hbm.at[pl.ds(upd_base + w * _W, _W)], buf, sem)

        def val_copy(w, buf, sem):
            return pltpu.make_async_copy(
                vals_hbm.at[pl.ds(upd_base + w * _W, _W)], buf, sem)

        idx_copy(0, idx0, sem_i0).start()
        idx_copy(1, idx1, sem_i1).start()

        @pl.loop(0, _W, step=16)
        def _(i):
            val0[pl.ds(i, 16)] = jnp.zeros((16,), jnp.float32)

        @pl.loop(0, _ACC_SLICE, step=_W)
        def _(j):
            pltpu.async_copy(val0, acc_sh.at[pl.ds(acc_base + j, _W)],
                             sem_s0)

        @pl.loop(0, _ACC_SLICE, step=_W)
        def _(j):
            pltpu.make_async_copy(
                val0, acc_sh.at[pl.ds(acc_base + j, _W)], sem_s0).wait()

        @pl.when(cid == 0)
        def _():
            val_copy(0, val0, sem_v0).start()
            val_copy(1, val1, sem_v1).start()

        @pl.when(cid == 1)
        def _():
            @pl.loop(0, _W, step=16)
            def _(i):
                val0[pl.ds(i, 16)] = jnp.full((16,), 1.0, jnp.float32)

        plsc.subcore_barrier()

        @pl.loop(0, _N_WIN, step=2)
        def _(w):
            idx_copy(w, idx0, sem_i0).wait()

            @pl.when(cid == 0)
            def _():
                val_copy(w, val0, sem_v0).wait()
                pltpu.sync_copy(val0, acc_sh.at[idx0], add=True)

            @pl.when(cid == 1)
            def _():
                pltpu.sync_copy(val0, acc_sh.at[idx0], add=True)

            @pl.when(w + 2 < _N_WIN)
            def _():
                idx_copy(w + 2, idx0, sem_i0).start()

                @pl.when(cid == 0)
                def _():
                    val_copy(w + 2, val0, sem_v0).start()

            idx_copy(w + 1, idx1, sem_i1).wait()

            @pl.when(cid == 0)
            def _():
                val_copy(w + 1, val1, sem_v1).wait()
                pltpu.sync_copy(val1, acc_sh.at[idx1], add=True)

            @pl.when(cid == 1)
            def _():
                pltpu.sync_copy(val0, acc_sh.at[idx1], add=True)

            @pl.when(w + 3 < _N_WIN)
            def _():
                idx_copy(w + 3, idx1, sem_i1).start()

                @pl.when(cid == 0)
                def _():
                    val_copy(w + 3, val1, sem_v1).start()

        plsc.subcore_barrier()

        acc_slc = pl.ds(acc_base, _ACC_SLICE)

        @pl.when(cid == 0)
        def _():
            pltpu.sync_copy(acc_sh.at[acc_slc], sums_hbm.at[acc_slc])

        @pl.when(cid == 1)
        def _():
            pltpu.sync_copy(acc_sh.at[acc_slc], counts_hbm.at[acc_slc])

    return scatter_kernel(node_strengths, node_indices)


def _combine_body(s_ref, sum_ref, cnt_ref, o_ref):
    o_ref[...] = s_ref[...] + sum_ref[...] / jnp.maximum(cnt_ref[...], 1.0)


def _combine(strengths, sums, counts):
    blk = 131072
    return pl.pallas_call(
        _combine_body,
        out_shape=jax.ShapeDtypeStruct((_N_NODES,), jnp.float32),
        grid=(8,),
        in_specs=[pl.BlockSpec((blk,), lambda i: (i,))] * 3,
        out_specs=pl.BlockSpec((blk,), lambda i: (i,)),
    )(strengths, sums, counts)


def kernel(node_strengths, node_indices, strengths):
    sums, counts = _sc_scatter(node_strengths, node_indices)
    return _combine(strengths, sums, counts)


# speedup vs baseline: 1.2041x; 1.0025x over previous
import functools

import jax
import jax.numpy as jnp
from jax import lax
from jax.experimental import pallas as pl
from jax.experimental.pallas import tpu as pltpu
from jax.experimental.pallas import tpu_sc as plsc

_N_NODES = 1000000
_N_UPD = 4194304
_ACC = 1 << 20
_N_TILES = 16
_W = 4096
_PER_TILE = _N_UPD // _N_TILES
_N_WIN = _PER_TILE // _W
_ACC_SLICE = _ACC // _N_TILES


def _sc_scatter(node_strengths, node_indices):
    mesh = plsc.VectorSubcoreMesh(core_axis_name="c", subcore_axis_name="s")

    @functools.partial(
        pl.kernel,
        out_type=(
            jax.ShapeDtypeStruct((_ACC,), jnp.float32),
            jax.ShapeDtypeStruct((_ACC,), jnp.float32),
        ),
        mesh=mesh,
        scratch_types=[
            pltpu.VMEM_SHARED((_ACC,), jnp.float32),
            pltpu.VMEM((_W,), jnp.int32),
            pltpu.VMEM((_W,), jnp.int32),
            pltpu.VMEM((_W,), jnp.float32),
            pltpu.VMEM((_W,), jnp.float32),
            pltpu.SemaphoreType.DMA,
            pltpu.SemaphoreType.DMA,
            pltpu.SemaphoreType.DMA,
            pltpu.SemaphoreType.DMA,
            pltpu.SemaphoreType.DMA,
            pltpu.SemaphoreType.DMA,
        ],
    )
    def scatter_kernel(vals_hbm, idx_hbm, sums_hbm, counts_hbm,
                       acc_sh, idx0, idx1, val0, val1,
                       sem_i0, sem_i1, sem_v0, sem_v1, sem_s0, sem_s1):
        cid = lax.axis_index("c")
        sid = lax.axis_index("s")
        upd_base = sid * _PER_TILE
        acc_base = sid * _ACC_SLICE

        def idx_copy(w, buf, sem):
            return pltpu.make_async_copy(
                idx_hbm.at[pl.ds(upd_base + w * _W, _W)], buf, sem)

        def val_copy(w, buf, sem):
            return pltpu.make_async_copy(
                vals_hbm.at[pl.ds(upd_base + w * _W, _W)], buf, sem)

        idx_copy(0, idx0, sem_i0).start()
        idx_copy(1, idx1, sem_i1).start()

        @pl.loop(0, _W, step=16)
        def _(i):
            val0[pl.ds(i, 16)] = jnp.zeros((16,), jnp.float32)

        @pl.loop(0, _ACC_SLICE, step=_W)
        def _(j):
            pltpu.async_copy(val0, acc_sh.at[pl.ds(acc_base + j, _W)],
                             sem_s0)

        @pl.loop(0, _ACC_SLICE, step=_W)
        def _(j):
            pltpu.make_async_copy(
                val0, acc_sh.at[pl.ds(acc_base + j, _W)], sem_s0).wait()

        @pl.when(cid == 0)
        def _():
            val_copy(0, val0, sem_v0).start()
            val_copy(1, val1, sem_v1).start()

        @pl.when(cid == 1)
        def _():
            @pl.loop(0, _W, step=16)
            def _(i):
                val0[pl.ds(i, 16)] = jnp.full((16,), 1.0, jnp.float32)

        plsc.subcore_barrier()

        @pl.loop(0, _N_WIN, step=2)
        def _(w):
            idx_copy(w, idx0, sem_i0).wait()

            @pl.when(cid == 0)
            def _():
                val_copy(w, val0, sem_v0).wait()
                pltpu.sync_copy(val0, acc_sh.at[idx0], add=True)

            @pl.when(cid == 1)
            def _():
                pltpu.sync_copy(val0, acc_sh.at[idx0], add=True)

            @pl.when(w + 2 < _N_WIN)
            def _():
                idx_copy(w + 2, idx0, sem_i0).start()

                @pl.when(cid == 0)
                def _():
                    val_copy(w + 2, val0, sem_v0).start()

            idx_copy(w + 1, idx1, sem_i1).wait()

            @pl.when(cid == 0)
            def _():
                val_copy(w + 1, val1, sem_v1).wait()
                pltpu.sync_copy(val1, acc_sh.at[idx1], add=True)

            @pl.when(cid == 1)
            def _():
                pltpu.sync_copy(val0, acc_sh.at[idx1], add=True)

            @pl.when(w + 3 < _N_WIN)
            def _():
                idx_copy(w + 3, idx1, sem_i1).start()

                @pl.when(cid == 0)
                def _():
                    val_copy(w + 3, val1, sem_v1).start()

        plsc.subcore_barrier()

        acc_slc = pl.ds(acc_base, _ACC_SLICE)

        @pl.when(cid == 0)
        def _():
            pltpu.sync_copy(acc_sh.at[acc_slc], sums_hbm.at[acc_slc])

        @pl.when(cid == 1)
        def _():
            pltpu.sync_copy(acc_sh.at[acc_slc], counts_hbm.at[acc_slc])

    return scatter_kernel(node_strengths, node_indices)


def _combine_body(s_ref, sum_ref, cnt_ref, o_ref):
    o_ref[...] = s_ref[...] + sum_ref[...] / jnp.maximum(cnt_ref[...], 1.0)


def _combine(strengths, sums, counts):
    blk = 131072
    return pl.pallas_call(
        _combine_body,
        out_shape=jax.ShapeDtypeStruct((_N_NODES,), jnp.float32),
        grid=(8,),
        in_specs=[pl.BlockSpec((blk,), lambda i: (i,))] * 3,
        out_specs=pl.BlockSpec((blk,), lambda i: (i,)),
    )(strengths, sums, counts)


def kernel(node_strengths, node_indices, strengths):
    sums, counts = _sc_scatter(node_strengths, node_indices)
    return _combine(strengths, sums, counts)
